# baseline (speedup 1.0000x reference)
import jax
import jax.numpy as jnp
from jax import lax
from jax.experimental import pallas as pl
from jax.experimental.pallas import tpu as pltpu
from jax.experimental.pallas import tpu_sc as plsc

_B = 32
_C = 3
_H = 224
_W = 224
_RCHUNK = 56
_NCH = _H // _RCHUNK


def _sc_body(tsb_hbm, sa_hbm, so_hbm, x0_hbm, n_hbm, out_hbm,
             idx16, sa16, so16, xb, nb, ob, semx, semn, semo):
    w = lax.axis_index("s") * 2 + lax.axis_index("c")
    pltpu.sync_copy(tsb_hbm.at[w], idx16)
    g1 = pltpu.async_copy(sa_hbm.at[idx16], sa16, semx.at[0])
    g2 = pltpu.async_copy(so_hbm.at[idx16], so16, semn.at[0])
    g1.wait()
    g2.wait()
    sa_v = sa16[...]
    so_v = so16[...]

    nchunks = _C * _NCH

    def _src(k):
        ch, r = divmod(k, _NCH)
        return (w, ch, pl.ds(r * _RCHUNK, _RCHUNK))

    in_flight = {}
    out_flight = {}

    def _start_in(k):
        s = k % 2
        in_flight[k] = (
            pltpu.async_copy(x0_hbm.at[_src(k)], xb.at[s], semx.at[s]),
            pltpu.async_copy(n_hbm.at[_src(k)], nb.at[s], semn.at[s]),
        )

    _start_in(0)
    for k in range(nchunks):
        s = k % 2
        if k + 1 < nchunks:
            _start_in(k + 1)
        cx, cn = in_flight.pop(k)
        cx.wait()
        cn.wait()
        if k >= 2:
            out_flight.pop(k - 2).wait()

        def _row(i, carry):
            for j in range(_W // 16):
                sl = pl.ds(j * 16, 16)
                ob[s, i, sl] = sa_v * xb[s, i, sl] + so_v * nb[s, i, sl]
            return carry

        lax.fori_loop(0, _RCHUNK, _row, 0)
        out_flight[k] = pltpu.async_copy(ob.at[s], out_hbm.at[_src(k)],
                                         semo.at[s])
    out_flight.pop(nchunks - 2).wait()
    out_flight.pop(nchunks - 1).wait()


@jax.jit
def kernel(x0, noise, time_steps, sqrt_alpha_cumprod, sqrt_one_minus_alpha_cumprod):
    ts_b = jnp.broadcast_to(time_steps.astype(jnp.int32)[:, None], (_B, 16))
    mesh = plsc.VectorSubcoreMesh(core_axis_name="c", subcore_axis_name="s")
    return pl.kernel(
        _sc_body,
        out_type=jax.ShapeDtypeStruct((_B, _C, _H, _W), jnp.float32),
        mesh=mesh,
        scratch_types=(
            pltpu.VMEM((16,), jnp.int32),
            pltpu.VMEM((16,), jnp.float32),
            pltpu.VMEM((16,), jnp.float32),
            pltpu.VMEM((2, _RCHUNK, _W), jnp.float32),
            pltpu.VMEM((2, _RCHUNK, _W), jnp.float32),
            pltpu.VMEM((2, _RCHUNK, _W), jnp.float32),
            pltpu.SemaphoreType.DMA((2,)),
            pltpu.SemaphoreType.DMA((2,)),
            pltpu.SemaphoreType.DMA((2,)),
        ),
    )(ts_b, sqrt_alpha_cumprod, sqrt_one_minus_alpha_cumprod, x0, noise)


# speedup vs baseline: 1.7873x; 1.7873x over previous
import jax
import jax.numpy as jnp
from jax import lax
from jax.experimental import pallas as pl
from jax.experimental.pallas import tpu as pltpu
from jax.experimental.pallas import tpu_sc as plsc

_B = 32


def _sc_gather_body(ts_hbm, sa_hbm, so_hbm, out_hbm,
                    idx_v, out_v, sem, sem2):
    sid = lax.axis_index("s")

    @pl.when(sid == 0)
    def _():
        pltpu.sync_copy(ts_hbm, idx_v)
        g1 = pltpu.async_copy(sa_hbm.at[idx_v], out_v.at[pl.ds(0, _B)], sem)
        g2 = pltpu.async_copy(so_hbm.at[idx_v], out_v.at[pl.ds(_B, _B)], sem2)
        g1.wait()
        g2.wait()
        pltpu.sync_copy(out_v, out_hbm)


def _sc_gather(time_steps, sa_table, so_table):
    mesh = plsc.VectorSubcoreMesh(core_axis_name="c", subcore_axis_name="s",
                                  num_cores=1, num_subcores=1)
    return pl.kernel(
        _sc_gather_body,
        out_type=jax.ShapeDtypeStruct((2 * _B,), jnp.float32),
        mesh=mesh,
        scratch_types=(
            pltpu.VMEM((_B,), jnp.int32),
            pltpu.VMEM((2 * _B,), jnp.float32),
            pltpu.SemaphoreType.DMA,
            pltpu.SemaphoreType.DMA,
        ),
    )(time_steps, sa_table, so_table)


_SAMPLES_PER_BLK = 4


def _tc_combine_body(scal_ref, x0_ref, n_ref, o_ref):
    g = pl.program_id(0)
    for i in range(_SAMPLES_PER_BLK):
        b = g * _SAMPLES_PER_BLK + i
        o_ref[i] = scal_ref[b] * x0_ref[i] + scal_ref[_B + b] * n_ref[i]


def _tc_combine(scal, x0, noise):
    s = _SAMPLES_PER_BLK
    grid = (_B // s,)
    c, h, w = x0.shape[1:]
    blk = pl.BlockSpec((s, c, h, w), lambda g: (g, 0, 0, 0))
    return pl.pallas_call(
        _tc_combine_body,
        grid=grid,
        in_specs=[
            pl.BlockSpec(memory_space=pltpu.SMEM),
            blk,
            blk,
        ],
        out_specs=blk,
        out_shape=jax.ShapeDtypeStruct(x0.shape, jnp.float32),
    )(scal, x0, noise)


@jax.jit
def kernel(x0, noise, time_steps, sqrt_alpha_cumprod, sqrt_one_minus_alpha_cumprod):
    ts = time_steps.astype(jnp.int32)
    scal = _sc_gather(ts, sqrt_alpha_cumprod, sqrt_one_minus_alpha_cumprod)
    return _tc_combine(scal, x0, noise)
